# baseline (speedup 1.0000x reference)
import functools

import jax
import jax.numpy as jnp
from jax import lax
from jax.experimental import pallas as pl
from jax.experimental.pallas import tpu as pltpu
from jax.experimental.pallas import tpu_sc as plsc

N = 16384
L = 200
D = 32
B = N * L

_info = plsc.get_sparse_core_info()
NC = _info.num_cores
NS = _info.num_subcores
NW = NC * NS
LT = L // 8
NT = N // 128
BLOCKS = LT * NT
BPW = BLOCKS // NW


@functools.partial(
    pl.kernel,
    mesh=plsc.VectorSubcoreMesh(core_axis_name="c", subcore_axis_name="s"),
    out_type=jax.ShapeDtypeStruct((L, 4, NT, 8, 128), jnp.float32),
    scratch_types=[
        pltpu.VMEM((1024,), jnp.int32),
        pltpu.VMEM((1024,), jnp.int32),
        pltpu.VMEM((1024, 32), jnp.float32),
        pltpu.VMEM((1024, 32), jnp.float32),
        pltpu.VMEM((8, 32, 129), jnp.float32),
        pltpu.SemaphoreType.DMA,
        pltpu.SemaphoreType.DMA,
        pltpu.SemaphoreType.DMA,
    ],
    compiler_params=pltpu.CompilerParams(
        use_tc_tiling_on_sc=False, needs_layout_passes=False),
)
def _gather_kernel(x5_hbm, table_hbm, out5_hbm, idx0, idx1, rows0, rows1,
                   tbuf, gsem, ssem, isem):
    idxs = (idx0, idx1)
    rows = (rows0, rows1)
    wid = lax.axis_index("s") * NC + lax.axis_index("c")
    base = wid * BPW
    iota16 = lax.iota(jnp.int32, 16)
    dhv = [dh * 16 + iota16 for dh in range(2)]

    for b in range(2):
        gid = base + b
        lt, nt = gid // NT, gid % NT
        pltpu.sync_copy(x5_hbm.at[lt, nt], idxs[b])
        pltpu.async_copy(
            table_hbm.at[idxs[b]], rows[b], gsem)

    def group(g, carry):
        for b in range(2):
            k = g * 2 + b
            gid = base + k
            lt, nt = gid // NT, gid % NT

            @pl.when(k > 0)
            def _():
                for dq in range(4):
                    pltpu.make_async_copy(
                        tbuf.at[pl.ds(0, 8), pl.ds(0, 8), pl.ds(0, 128)],
                        out5_hbm.at[pl.ds(0, 8), 0, 0], ssem).wait()

            pltpu.make_async_copy(
                table_hbm.at[idxs[b]], rows[b],
                gsem).wait()

            @pl.when(k < BPW - 2)
            def _():
                gid2 = gid + 2
                pltpu.async_copy(
                    x5_hbm.at[gid2 // NT, gid2 % NT], idxs[b], isem)

            def slloop(sl, c, b=b, lt=lt, nt=nt):
                tsl = tbuf.at[sl]

                @plsc.parallel_loop(0, 128, step=1, unroll=4)
                def lnloop(ln, b=b, sl=sl, tsl=tsl):
                    row = sl * 128 + ln
                    lnsplat = jnp.full((16,), ln, jnp.int32)
                    for dh in range(2):
                        v = rows[b][row, pl.ds(dh * 16, 16)]
                        plsc.store_scatter(tsl, [dhv[dh], lnsplat], v)

                l = lt * 8 + sl
                for dt in range(4):
                    pltpu.async_copy(
                        tbuf.at[sl, pl.ds(dt * 8, 8), pl.ds(0, 128)],
                        out5_hbm.at[l, dt, nt], ssem)
                return c

            lax.fori_loop(0, 8, slloop, 0)

            @pl.when(k < BPW - 2)
            def _():
                gid2 = gid + 2
                pltpu.make_async_copy(
                    x5_hbm.at[gid2 // NT, gid2 % NT], idxs[b], isem).wait()
                pltpu.async_copy(
            table_hbm.at[idxs[b]], rows[b], gsem)
        return carry

    lax.fori_loop(0, BPW // 2, group, 0)

    for dq in range(4):
        pltpu.make_async_copy(
            tbuf.at[pl.ds(0, 8), pl.ds(0, 8), pl.ds(0, 128)],
            out5_hbm.at[pl.ds(0, 8), 0, 0], ssem).wait()


V = 1000000
TC = 768
TCHUNKS = 41


@functools.partial(
    pl.kernel,
    mesh=plsc.VectorSubcoreMesh(core_axis_name="c", subcore_axis_name="s"),
    out_type=jax.ShapeDtypeStruct((V, D), jnp.float32),
    scratch_types=[
        pltpu.VMEM((D, TC), jnp.float32),
        pltpu.VMEM((D, TC), jnp.float32),
        pltpu.VMEM((TC, 33), jnp.float32),
        pltpu.VMEM((TC, 33), jnp.float32),
        pltpu.SemaphoreType.DMA,
        pltpu.SemaphoreType.DMA,
    ],
    compiler_params=pltpu.CompilerParams(
        use_tc_tiling_on_sc=False, needs_layout_passes=False),
)
def _table_kernel(tt_hbm, out_hbm, a0, a1, t0, t1, asem, osem):
    abuf = (a0, a1)
    tbta = (t0, t1)
    wid = lax.axis_index("s") * NC + lax.axis_index("c")
    start = (wid * (V // NW)) // 16 * 16
    end = jnp.minimum(((wid + 1) * (V // NW) + 15) // 16 * 16, V)
    last = end - TC - start
    iota16 = lax.iota(jnp.int32, 16)
    fsplat = [jnp.full((16,), f, jnp.int32) for f in range(D)]

    def src_off(j):
        return start + jnp.minimum(j * TC, last)

    for b in range(2):
        pltpu.async_copy(
            tt_hbm.at[:, pl.ds(src_off(b), TC)], abuf[b], asem)

    def group(g, carry):
        for b in range(2):
            j = g * 2 + b
            r0 = src_off(j)
            pltpu.make_async_copy(
                tt_hbm.at[:, pl.ds(r0, TC)], abuf[b], asem).wait()

            @pl.when(j > 1)
            def _():
                pltpu.make_async_copy(
                    tbta[b].at[pl.ds(0, TC), pl.ds(0, 32)],
                    out_hbm.at[pl.ds(0, TC)], osem).wait()

            @plsc.parallel_loop(0, TC // 16, step=1, unroll=4)
            def vloop(vg, b=b):
                vvec = iota16 + vg * 16
                for f in range(D):
                    v = abuf[b][f, pl.ds(vg * 16, 16)]
                    plsc.store_scatter(tbta[b], [vvec, fsplat[f]], v)

            pltpu.async_copy(
                tbta[b].at[pl.ds(0, TC), pl.ds(0, 32)],
                out_hbm.at[pl.ds(r0, TC)], osem)

            @pl.when(j < TCHUNKS - 2)
            def _():
                pltpu.async_copy(
                    tt_hbm.at[:, pl.ds(src_off(j + 2), TC)], abuf[b], asem)
        return carry

    lax.fori_loop(0, TCHUNKS // 2, group, 0)
    j = TCHUNKS - 1
    r0 = src_off(j)
    pltpu.make_async_copy(
        tt_hbm.at[:, pl.ds(r0, TC)], abuf[0], asem).wait()
    pltpu.make_async_copy(
        tbta[0].at[pl.ds(0, TC), pl.ds(0, 32)],
        out_hbm.at[pl.ds(0, TC)], osem).wait()

    @plsc.parallel_loop(0, TC // 16, step=1, unroll=4)
    def vloop2(vg):
        vvec = iota16 + vg * 16
        for f in range(D):
            v = abuf[0][f, pl.ds(vg * 16, 16)]
            plsc.store_scatter(tbta[0], [vvec, fsplat[f]], v)

    pltpu.async_copy(
        tbta[0].at[pl.ds(0, TC), pl.ds(0, 32)],
        out_hbm.at[pl.ds(r0, TC)], osem)
    for b in range(2):
        pltpu.make_async_copy(
            tbta[b].at[pl.ds(0, TC), pl.ds(0, 32)],
            out_hbm.at[pl.ds(0, TC)], osem).wait()


def kernel(x, embeddings):
    x5 = (x.astype(jnp.int32).reshape(128, 128, LT, 8)
         .transpose(2, 0, 3, 1).reshape(LT, NT, 1024))
    table_lin = _table_kernel(embeddings.T)
    out5 = _gather_kernel(x5, table_lin)
    return out5.transpose(2, 4, 0, 1, 3).reshape(N, L, D)


# speedup vs baseline: 3.8770x; 3.8770x over previous
import functools

import jax
import jax.numpy as jnp
from jax import lax
from jax.experimental import pallas as pl
from jax.experimental.pallas import tpu as pltpu
from jax.experimental.pallas import tpu_sc as plsc

N = 16384
L = 200
D = 32
B = N * L

_info = plsc.get_sparse_core_info()
NC = _info.num_cores
NS = _info.num_subcores
NW = NC * NS
LT = L // 8
NT = N // 128
BLOCKS = LT * NT
BPW = BLOCKS // NW


@functools.partial(
    pl.kernel,
    mesh=plsc.VectorSubcoreMesh(core_axis_name="c", subcore_axis_name="s"),
    out_type=jax.ShapeDtypeStruct((L, 4, NT, 8, 128), jnp.float32),
    scratch_types=[
        pltpu.VMEM((1024,), jnp.int32),
        pltpu.VMEM((1024,), jnp.int32),
        pltpu.VMEM((1024, 32), jnp.float32),
        pltpu.VMEM((1024, 32), jnp.float32),
        pltpu.VMEM((8, 32, 129), jnp.float32),
        pltpu.SemaphoreType.DMA,
        pltpu.SemaphoreType.DMA,
        pltpu.SemaphoreType.DMA,
    ],
    compiler_params=pltpu.CompilerParams(
        use_tc_tiling_on_sc=False, needs_layout_passes=False),
)
def _gather_kernel(x5_hbm, table_hbm, out5_hbm, idx0, idx1, rows0, rows1,
                   tbuf, gsem, ssem, isem):
    idxs = (idx0, idx1)
    rows = (rows0, rows1)
    wid = lax.axis_index("s") * NC + lax.axis_index("c")
    base = wid * BPW
    iota16 = lax.iota(jnp.int32, 16)
    dhv = [dh * 16 + iota16 for dh in range(2)]

    for b in range(2):
        gid = base + b
        lt, nt = gid // NT, gid % NT
        pltpu.sync_copy(x5_hbm.at[lt, nt], idxs[b])
        pltpu.async_copy(
            table_hbm.at[idxs[b]], rows[b], gsem)

    def group(g, carry):
        for b in range(2):
            k = g * 2 + b
            gid = base + k
            lt, nt = gid // NT, gid % NT

            @pl.when(k > 0)
            def _():
                for dq in range(4):
                    pltpu.make_async_copy(
                        tbuf.at[pl.ds(0, 8), pl.ds(0, 8), pl.ds(0, 128)],
                        out5_hbm.at[pl.ds(0, 8), 0, 0], ssem).wait()

            pltpu.make_async_copy(
                table_hbm.at[idxs[b]], rows[b],
                gsem).wait()

            @pl.when(k < BPW - 2)
            def _():
                gid2 = gid + 2
                pltpu.async_copy(
                    x5_hbm.at[gid2 // NT, gid2 % NT], idxs[b], isem)

            def slloop(sl, c, b=b, lt=lt, nt=nt):
                tsl = tbuf.at[sl]

                @plsc.parallel_loop(0, 128, step=1, unroll=4)
                def lnloop(ln, b=b, sl=sl, tsl=tsl):
                    row = sl * 128 + ln
                    lnsplat = jnp.full((16,), ln, jnp.int32)
                    for dh in range(2):
                        v = rows[b][row, pl.ds(dh * 16, 16)]
                        plsc.store_scatter(tsl, [dhv[dh], lnsplat], v)

                l = lt * 8 + sl
                for dt in range(4):
                    pltpu.async_copy(
                        tbuf.at[sl, pl.ds(dt * 8, 8), pl.ds(0, 128)],
                        out5_hbm.at[l, dt, nt], ssem)
                return c

            lax.fori_loop(0, 8, slloop, 0)

            @pl.when(k < BPW - 2)
            def _():
                gid2 = gid + 2
                pltpu.make_async_copy(
                    x5_hbm.at[gid2 // NT, gid2 % NT], idxs[b], isem).wait()
                pltpu.async_copy(
            table_hbm.at[idxs[b]], rows[b], gsem)
        return carry

    lax.fori_loop(0, BPW // 2, group, 0)

    for dq in range(4):
        pltpu.make_async_copy(
            tbuf.at[pl.ds(0, 8), pl.ds(0, 8), pl.ds(0, 128)],
            out5_hbm.at[pl.ds(0, 8), 0, 0], ssem).wait()


def kernel(x, embeddings):
    x5 = (x.astype(jnp.int32).reshape(128, 128, LT, 8)
         .transpose(2, 0, 3, 1).reshape(LT, NT, 1024))
    table128 = lax.optimization_barrier(embeddings.reshape(250000, 128))
    out5 = _gather_kernel(x5, table128.reshape(1000000, 32))
    return out5.transpose(2, 4, 0, 1, 3).reshape(N, L, D)


# speedup vs baseline: 3.8805x; 1.0009x over previous
import functools

import jax
import jax.numpy as jnp
from jax import lax
from jax.experimental import pallas as pl
from jax.experimental.pallas import tpu as pltpu
from jax.experimental.pallas import tpu_sc as plsc

N = 16384
L = 200
D = 32
B = N * L

_info = plsc.get_sparse_core_info()
NC = _info.num_cores
NS = _info.num_subcores
NW = NC * NS
LT = L // 8
NT = N // 128
BLOCKS = LT * NT
BPW = BLOCKS // NW


@functools.partial(
    pl.kernel,
    mesh=plsc.VectorSubcoreMesh(core_axis_name="c", subcore_axis_name="s"),
    out_type=jax.ShapeDtypeStruct((L, 4, NT, 8, 128), jnp.float32),
    scratch_types=[
        pltpu.VMEM((1024,), jnp.int32),
        pltpu.VMEM((1024,), jnp.int32),
        pltpu.VMEM((1024, 32), jnp.float32),
        pltpu.VMEM((1024, 32), jnp.float32),
        pltpu.VMEM((8, 32, 129), jnp.float32),
        pltpu.SemaphoreType.DMA,
        pltpu.SemaphoreType.DMA,
        pltpu.SemaphoreType.DMA,
    ],
    compiler_params=pltpu.CompilerParams(
        use_tc_tiling_on_sc=False, needs_layout_passes=False),
)
def _gather_kernel(x5_hbm, table_hbm, out5_hbm, idx0, idx1, rows0, rows1,
                   tbuf, gsem, ssem, isem):
    idxs = (idx0, idx1)
    rows = (rows0, rows1)
    wid = lax.axis_index("s") * NC + lax.axis_index("c")
    base = wid * BPW
    iota16 = lax.iota(jnp.int32, 16)
    dhv = [dh * 16 + iota16 for dh in range(2)]

    for b in range(2):
        gid = base + b
        lt, nt = gid // NT, gid % NT
        pltpu.sync_copy(x5_hbm.at[lt, nt], idxs[b])
        pltpu.async_copy(
            table_hbm.at[idxs[b]], rows[b], gsem)

    def group(g, carry):
        for b in range(2):
            k = g * 2 + b
            gid = base + k
            lt, nt = gid // NT, gid % NT

            @pl.when(k > 0)
            def _():
                for dq in range(4):
                    pltpu.make_async_copy(
                        tbuf.at[pl.ds(0, 8), pl.ds(0, 8), pl.ds(0, 128)],
                        out5_hbm.at[pl.ds(0, 8), 0, 0], ssem).wait()

            pltpu.make_async_copy(
                table_hbm.at[idxs[b]], rows[b],
                gsem).wait()

            @pl.when(k < BPW - 2)
            def _():
                gid2 = gid + 2
                pltpu.async_copy(
                    x5_hbm.at[gid2 // NT, gid2 % NT], idxs[b], isem)

            def slloop(sl, c, b=b, lt=lt, nt=nt):
                tsl = tbuf.at[sl]

                @plsc.parallel_loop(0, 128, step=1, unroll=4)
                def lnloop(ln, b=b, sl=sl, tsl=tsl):
                    row = sl * 128 + ln
                    lnsplat = jnp.full((16,), ln, jnp.int32)
                    for dh in range(2):
                        v = rows[b][row, pl.ds(dh * 16, 16)]
                        plsc.store_scatter(tsl, [dhv[dh], lnsplat], v)

                l = lt * 8 + sl
                for dt in range(4):
                    pltpu.async_copy(
                        tbuf.at[sl, pl.ds(dt * 8, 8), pl.ds(0, 128)],
                        out5_hbm.at[l, dt, nt], ssem)
                return c

            lax.fori_loop(0, 8, slloop, 0)

            @pl.when(k < BPW - 2)
            def _():
                gid2 = gid + 2
                pltpu.make_async_copy(
                    x5_hbm.at[gid2 // NT, gid2 % NT], idxs[b], isem).wait()
                pltpu.async_copy(
            table_hbm.at[idxs[b]], rows[b], gsem)
        return carry

    lax.fori_loop(0, BPW // 2, group, 0)

    for dq in range(4):
        pltpu.make_async_copy(
            tbuf.at[pl.ds(0, 8), pl.ds(0, 8), pl.ds(0, 128)],
            out5_hbm.at[pl.ds(0, 8), 0, 0], ssem).wait()


def kernel(x, embeddings):
    x5 = (x.astype(jnp.int32).reshape(128, 128, LT, 8)
         .transpose(2, 0, 3, 1).reshape(LT, NT, 1024))
    out5 = _gather_kernel(x5, embeddings)
    return out5.transpose(2, 4, 0, 1, 3).reshape(N, L, D)
